# baseline (device time: 39000 ns/iter reference)
import jax
import jax.numpy as jnp
from jax import lax
from jax.experimental import pallas as pl
from jax.experimental.pallas import tpu as pltpu

N_DEV = 4
WINDOW = 128


def kernel(x, Wq, K_ext, V_ext, Wo):
    B, Sq, D = x.shape
    _, Skv, Hl, Dh = K_ext.shape
    Dl = Hl * Dh
    Qr = Sq // 2
    KW = Qr + WINDOW

    def body(x_ref, wq_ref, k_ref, v_ref, wo_ref, out_ref,
             ctx_ref, qb_ref, kk_ref, vv_ref, s1_ref, r1a_ref, r1b_ref,
             s2_ref, r2_ref, s3_ref, r3_ref, r4a_ref, r4b_ref,
             send_sems, recv_sems):
        my = lax.axis_index("i")
        a1 = jnp.bitwise_and(jnp.bitwise_xor(my, my >> 1), 1)
        a2 = jnp.bitwise_and(my >> 1, 1)
        p1 = jnp.bitwise_xor(my, 1)
        p2 = 3 - my
        bs = 1 - a1

        barrier_sem = pltpu.get_barrier_semaphore()
        for nbr in (p1, p2):
            pl.semaphore_signal(
                barrier_sem, inc=1,
                device_id=(nbr,), device_id_type=pl.DeviceIdType.MESH,
            )
        pl.semaphore_wait(barrier_sem, 2)

        wq_loc = wq_ref[:, pl.ds(my * Dl, Dl)].astype(jnp.bfloat16)
        wo_loc = wo_ref[pl.ds(my * Dl, Dl), :].astype(jnp.bfloat16)

        def load_b(bd):
            xb = x_ref[bd].astype(jnp.bfloat16)
            qb = jnp.dot(xb, wq_loc, preferred_element_type=jnp.float32)
            qb_ref[...] = (qb * 0.125).astype(jnp.bfloat16)
            kk_ref[...] = k_ref[bd].astype(jnp.bfloat16).reshape(Sq, Dl)
            vv_ref[...] = v_ref[bd].astype(jnp.bfloat16).reshape(Sq, Dl)

        def chunk(bd, qh, s1_slot):
            k0 = qh * WINDOW
            band = (
                jnp.abs(
                    lax.broadcasted_iota(jnp.int32, (Qr, KW), 0)
                    - lax.broadcasted_iota(jnp.int32, (Qr, KW), 1)
                    + qh * WINDOW
                )
                <= WINDOW
            ).astype(jnp.float32)
            for h in range(Hl):
                q = qb_ref[pl.ds(qh * Qr, Qr), h * Dh:(h + 1) * Dh]
                k = kk_ref[pl.ds(k0, KW), h * Dh:(h + 1) * Dh]
                s = lax.dot_general(
                    q, k, (((1,), (1,)), ((), ())),
                    preferred_element_type=jnp.float32,
                )
                e = jnp.exp(s) * band
                denom = jnp.sum(e, axis=-1, keepdims=True)
                v = vv_ref[pl.ds(k0, KW), h * Dh:(h + 1) * Dh]
                ctx = lax.dot_general(
                    e.astype(jnp.bfloat16), v, (((1,), (0,)), ((), ())),
                    preferred_element_type=jnp.float32,
                )
                ctx_ref[:, h * Dh:(h + 1) * Dh] = (ctx / denom).astype(
                    jnp.bfloat16)
            pb = jnp.dot(ctx_ref[...], wo_loc,
                         preferred_element_type=jnp.float32)
            out_ref[bd, pl.ds(qh * Qr, Qr), :] = pb
            if s1_slot is not None:
                s1_ref[s1_slot] = pb.astype(jnp.bfloat16)

        mk = lambda src, dst, i, dev: pltpu.make_async_remote_copy(
            src_ref=src, dst_ref=dst,
            send_sem=send_sems.at[i], recv_sem=recv_sems.at[i],
            device_id=(dev,), device_id_type=pl.DeviceIdType.MESH,
        )
        rdma1a = mk(s1_ref.at[0], r1a_ref, 0, p1)
        rdma1b = mk(s1_ref.at[1], r1b_ref, 1, p1)
        rdma2 = mk(s2_ref, r2_ref, 2, p2)
        Hr = Qr // 2
        rdma3a = mk(s3_ref.at[pl.ds(0, Hr)], r3_ref.at[pl.ds(0, Hr)], 3, p2)
        rdma3b = mk(s3_ref.at[pl.ds(Hr, Hr)], r3_ref.at[pl.ds(Hr, Hr)], 6, p2)
        rdma4a = mk(s3_ref, r4a_ref, 4, p1)
        rdma4ba = mk(r3_ref.at[pl.ds(0, Hr)], r4b_ref.at[pl.ds(0, Hr)], 5, p1)
        rdma4bb = mk(r3_ref.at[pl.ds(Hr, Hr)], r4b_ref.at[pl.ds(Hr, Hr)], 7, p1)

        load_b(bs)
        chunk(bs, 1 - a2, 0)
        rdma1a.start()
        chunk(bs, a2, 1)
        rdma1b.start()

        load_b(a1)
        chunk(a1, 1 - a2, None)

        rdma1a.wait_recv()
        off_s = (1 - a2) * Qr
        out_ref[a1, pl.ds(off_s, Qr), :] = (
            out_ref[a1, pl.ds(off_s, Qr), :] + r1a_ref[...].astype(jnp.float32)
        )
        s2_ref[...] = out_ref[a1, pl.ds(off_s, Qr), :].astype(jnp.bfloat16)
        rdma2.start()

        chunk(a1, a2, None)

        off_k = a2 * Qr
        rdma1b.wait_recv()
        out_ref[a1, pl.ds(off_k, Qr), :] = (
            out_ref[a1, pl.ds(off_k, Qr), :] + r1b_ref[...].astype(jnp.float32)
        )
        rdma2.wait_recv()
        out_ref[a1, pl.ds(off_k, Qr), :] = (
            out_ref[a1, pl.ds(off_k, Qr), :] + r2_ref[...].astype(jnp.float32)
        )

        s3_ref[...] = out_ref[a1, pl.ds(off_k, Qr), :].astype(jnp.bfloat16)
        rdma4a.start()
        rdma3a.start()
        rdma3b.start()

        rdma3a.wait_recv()
        out_ref[a1, pl.ds(off_s, Hr), :] = r3_ref[0:Hr, :].astype(jnp.float32)
        rdma4ba.start()
        rdma3b.wait_recv()
        out_ref[a1, pl.ds(off_s + Hr, Hr), :] = r3_ref[Hr:Qr, :].astype(
            jnp.float32)
        rdma4bb.start()

        rdma4a.wait_recv()
        out_ref[bs, pl.ds(a2 * Qr, Qr), :] = r4a_ref[...].astype(jnp.float32)
        rdma4ba.wait_recv()
        out_ref[bs, pl.ds((1 - a2) * Qr, Hr), :] = r4b_ref[0:Hr, :].astype(
            jnp.float32)
        rdma4bb.wait_recv()
        out_ref[bs, pl.ds((1 - a2) * Qr + Hr, Hr), :] = r4b_ref[Hr:Qr, :].astype(
            jnp.float32)

        for r in (rdma1a, rdma1b, rdma2, rdma3a, rdma3b, rdma4a,
                  rdma4ba, rdma4bb):
            r.wait_send()

    return pl.pallas_call(
        body,
        out_shape=jax.ShapeDtypeStruct((B, Sq, D), jnp.float32),
        in_specs=[pl.BlockSpec(memory_space=pltpu.VMEM)] * 5,
        out_specs=pl.BlockSpec(memory_space=pltpu.VMEM),
        scratch_shapes=[
            pltpu.VMEM((Qr, Dl), jnp.bfloat16),
            pltpu.VMEM((Sq, Dl), jnp.bfloat16),
            pltpu.VMEM((Sq, Dl), jnp.bfloat16),
            pltpu.VMEM((Sq, Dl), jnp.bfloat16),
            pltpu.VMEM((2, Qr, D), jnp.bfloat16),
            pltpu.VMEM((Qr, D), jnp.bfloat16),
            pltpu.VMEM((Qr, D), jnp.bfloat16),
            pltpu.VMEM((Qr, D), jnp.bfloat16),
            pltpu.VMEM((Qr, D), jnp.bfloat16),
            pltpu.VMEM((Qr, D), jnp.bfloat16),
            pltpu.VMEM((Qr, D), jnp.bfloat16),
            pltpu.VMEM((Qr, D), jnp.bfloat16),
            pltpu.VMEM((Qr, D), jnp.bfloat16),
            pltpu.SemaphoreType.DMA((8,)),
            pltpu.SemaphoreType.DMA((8,)),
        ],
        compiler_params=pltpu.CompilerParams(collective_id=0),
    )(x, Wq, K_ext, V_ext, Wo)
